# baseline (speedup 1.0000x reference)
import functools

import jax
import jax.numpy as jnp
from jax import lax
from jax.experimental import pallas as pl
from jax.experimental.pallas import tpu as pltpu
from jax.experimental.pallas import tpu_sc as plsc

B, L, D = 16, 4096, 768
LANES = 16
NCOL = D // LANES
SC_ROWS = 2304
CHUNK = 64
NBUF = 2
SC_HALF = SC_ROWS // 2
NCH = SC_HALF // CHUNK
BL = 128
TC_NL = (L - SC_ROWS) // BL
TC_OFF = SC_ROWS // BL
assert SC_ROWS % (2 * CHUNK * NBUF) == 0 or SC_ROWS == 0
assert SC_ROWS % BL == 0

_mesh = plsc.VectorSubcoreMesh(core_axis_name="c", subcore_axis_name="s")


def _zero_acc(acc):
    def body(j, _):
        acc[pl.ds(j * LANES, LANES)] = jnp.zeros((LANES,), jnp.float32)
        return 0

    lax.fori_loop(0, NCOL, body, 0)


def _accumulate(buf, acc):
    def col_body(j, _):
        col = j * LANES
        z = jnp.zeros((LANES,), jnp.float32)

        def row_body(r, carry):
            s0, s1, s2, s3 = carry
            r4 = r * 4
            s0 = s0 + buf[r4, pl.ds(col, LANES)]
            s1 = s1 + buf[r4 + 1, pl.ds(col, LANES)]
            s2 = s2 + buf[r4 + 2, pl.ds(col, LANES)]
            s3 = s3 + buf[r4 + 3, pl.ds(col, LANES)]
            return (s0, s1, s2, s3)

        s0, s1, s2, s3 = lax.fori_loop(
            0, CHUNK // 4, row_body, (z, z, z, z), unroll=4
        )
        acc[pl.ds(col, LANES)] = acc[pl.ds(col, LANES)] + (
            (s0 + s1) + (s2 + s3)
        )
        return 0

    lax.fori_loop(0, NCOL, col_body, 0)


@functools.partial(
    pl.kernel,
    mesh=_mesh,
    out_type=jax.ShapeDtypeStruct((B, 1, D), jnp.float32),
    scratch_types=[
        *([pltpu.VMEM((CHUNK, D), jnp.float32)] * NBUF),
        pltpu.VMEM((D,), jnp.float32),
        pltpu.VMEM((2, D), jnp.float32),
        pltpu.VMEM_SHARED((16, D), jnp.float32),
        *([pltpu.SemaphoreType.DMA] * NBUF),
    ],
)
def _sc_mean(x_hbm, out_hbm, *refs):
    bufs = refs[:NBUF]
    acc, pair, shared = refs[NBUF : NBUF + 3]
    sems = refs[NBUF + 3 :]
    c = lax.axis_index("c")
    s = lax.axis_index("s")
    b = c * 8 + s // 2
    row0 = (s % 2) * SC_HALF

    _zero_acc(acc)

    for i in range(NBUF):
        pltpu.async_copy(
            x_hbm.at[b, pl.ds(row0 + i * CHUNK, CHUNK)], bufs[i], sems[i]
        )

    def outer(k, _):
        for i in range(NBUF):
            idx = NBUF * k + i
            base = row0 + idx * CHUNK
            pltpu.make_async_copy(
                x_hbm.at[b, pl.ds(base, CHUNK)], bufs[i], sems[i]
            ).wait()
            _accumulate(bufs[i], acc)

            @pl.when(idx + NBUF < NCH)
            def _():
                pltpu.async_copy(
                    x_hbm.at[b, pl.ds(base + NBUF * CHUNK, CHUNK)],
                    bufs[i],
                    sems[i],
                )

        return 0

    lax.fori_loop(0, NCH // NBUF, outer, 0)

    pltpu.sync_copy(acc, shared.at[s])
    plsc.subcore_barrier()

    @pl.when(s % 2 == 0)
    def _():
        pltpu.sync_copy(shared.at[s], pair.at[0])
        pltpu.sync_copy(shared.at[s + 1], pair.at[1])

        def fin(j, _):
            col = j * LANES
            m = (pair[0, pl.ds(col, LANES)] + pair[1, pl.ds(col, LANES)]) * (
                1.0 / L
            )
            acc[pl.ds(col, LANES)] = m
            return 0

        lax.fori_loop(0, NCOL, fin, 0)
        pltpu.sync_copy(acc, out_hbm.at[b, 0])


def _tc_body(x_ref, o_ref):
    li = pl.program_id(1)

    @pl.when(li == 0)
    def _():
        o_ref[...] = jnp.zeros_like(o_ref)

    o_ref[...] += jnp.sum(x_ref[...], axis=1, keepdims=True)

    @pl.when(li == pl.num_programs(1) - 1)
    def _():
        o_ref[...] = o_ref[...] * (1.0 / L)


def _tc_mean(x):
    return pl.pallas_call(
        _tc_body,
        grid=(B, TC_NL),
        in_specs=[
            pl.BlockSpec((1, BL, D), lambda b, l: (b, l + TC_OFF, 0)),
        ],
        out_specs=pl.BlockSpec((1, 1, D), lambda b, l: (b, 0, 0)),
        out_shape=jax.ShapeDtypeStruct((B, 1, D), jnp.float32),
        compiler_params=pltpu.CompilerParams(
            dimension_semantics=("parallel", "arbitrary"),
        ),
    )(x)


def kernel(x):
    if SC_ROWS == 0:
        return _tc_mean(x)
    if SC_ROWS == L:
        return _sc_mean(x)
    return _sc_mean(x) + _tc_mean(x)


# speedup vs baseline: 1.7343x; 1.7343x over previous
import functools

import jax
import jax.numpy as jnp
from jax import lax
from jax.experimental import pallas as pl
from jax.experimental.pallas import tpu as pltpu
from jax.experimental.pallas import tpu_sc as plsc

B, L, D = 16, 4096, 768
LANES = 16
NCOL = D // LANES
SC_ROWS = 2048
CHUNK = 64
NBUF = 2
SC_HALF = SC_ROWS // 2
NCH = SC_HALF // CHUNK
BL = 1024
TC_NL = (L - SC_ROWS) // BL
TC_OFF = SC_ROWS // BL
assert SC_ROWS % (2 * CHUNK * NBUF) == 0 or SC_ROWS == 0
assert SC_ROWS % BL == 0

_mesh = plsc.VectorSubcoreMesh(core_axis_name="c", subcore_axis_name="s")


def _zero_acc(acc):
    def body(j, _):
        acc[pl.ds(j * LANES, LANES)] = jnp.zeros((LANES,), jnp.float32)
        return 0

    lax.fori_loop(0, NCOL, body, 0)


def _accumulate(buf, acc):
    def col_body(j, _):
        col = j * LANES
        z = jnp.zeros((LANES,), jnp.float32)

        def row_body(r, carry):
            s0, s1, s2, s3 = carry
            r4 = r * 4
            s0 = s0 + buf[r4, pl.ds(col, LANES)]
            s1 = s1 + buf[r4 + 1, pl.ds(col, LANES)]
            s2 = s2 + buf[r4 + 2, pl.ds(col, LANES)]
            s3 = s3 + buf[r4 + 3, pl.ds(col, LANES)]
            return (s0, s1, s2, s3)

        s0, s1, s2, s3 = lax.fori_loop(
            0, CHUNK // 4, row_body, (z, z, z, z), unroll=4
        )
        acc[pl.ds(col, LANES)] = acc[pl.ds(col, LANES)] + (
            (s0 + s1) + (s2 + s3)
        )
        return 0

    lax.fori_loop(0, NCOL, col_body, 0)


@functools.partial(
    pl.kernel,
    mesh=_mesh,
    out_type=jax.ShapeDtypeStruct((B, 1, D), jnp.float32),
    scratch_types=[
        *([pltpu.VMEM((CHUNK, D), jnp.float32)] * NBUF),
        pltpu.VMEM((D,), jnp.float32),
        pltpu.VMEM((2, D), jnp.float32),
        pltpu.VMEM_SHARED((16, D), jnp.float32),
        *([pltpu.SemaphoreType.DMA] * NBUF),
    ],
)
def _sc_mean(x_hbm, out_hbm, *refs):
    bufs = refs[:NBUF]
    acc, pair, shared = refs[NBUF : NBUF + 3]
    sems = refs[NBUF + 3 :]
    c = lax.axis_index("c")
    s = lax.axis_index("s")
    b = c * 8 + s // 2
    row0 = (s % 2) * SC_HALF

    _zero_acc(acc)

    for i in range(NBUF):
        pltpu.async_copy(
            x_hbm.at[b, pl.ds(row0 + i * CHUNK, CHUNK)], bufs[i], sems[i]
        )

    def outer(k, _):
        for i in range(NBUF):
            idx = NBUF * k + i
            base = row0 + idx * CHUNK
            pltpu.make_async_copy(
                x_hbm.at[b, pl.ds(base, CHUNK)], bufs[i], sems[i]
            ).wait()
            _accumulate(bufs[i], acc)

            @pl.when(idx + NBUF < NCH)
            def _():
                pltpu.async_copy(
                    x_hbm.at[b, pl.ds(base + NBUF * CHUNK, CHUNK)],
                    bufs[i],
                    sems[i],
                )

        return 0

    lax.fori_loop(0, NCH // NBUF, outer, 0)

    pltpu.sync_copy(acc, shared.at[s])
    plsc.subcore_barrier()

    @pl.when(s % 2 == 0)
    def _():
        pltpu.sync_copy(shared.at[s], pair.at[0])
        pltpu.sync_copy(shared.at[s + 1], pair.at[1])

        def fin(j, _):
            col = j * LANES
            m = (pair[0, pl.ds(col, LANES)] + pair[1, pl.ds(col, LANES)]) * (
                1.0 / L
            )
            acc[pl.ds(col, LANES)] = m
            return 0

        lax.fori_loop(0, NCOL, fin, 0)
        pltpu.sync_copy(acc, out_hbm.at[b, 0])


def _tc_body(x_ref, o_ref):
    li = pl.program_id(1)

    @pl.when(li == 0)
    def _():
        o_ref[...] = jnp.zeros_like(o_ref)

    o_ref[...] += jnp.sum(x_ref[...], axis=1, keepdims=True)

    @pl.when(li == pl.num_programs(1) - 1)
    def _():
        o_ref[...] = o_ref[...] * (1.0 / L)


def _tc_mean(x):
    return pl.pallas_call(
        _tc_body,
        grid=(B, TC_NL),
        in_specs=[
            pl.BlockSpec((1, BL, D), lambda b, l: (b, l + TC_OFF, 0)),
        ],
        out_specs=pl.BlockSpec((1, 1, D), lambda b, l: (b, 0, 0)),
        out_shape=jax.ShapeDtypeStruct((B, 1, D), jnp.float32),
        compiler_params=pltpu.CompilerParams(
            dimension_semantics=("parallel", "arbitrary"),
        ),
    )(x)


def kernel(x):
    if SC_ROWS == 0:
        return _tc_mean(x)
    if SC_ROWS == L:
        return _sc_mean(x)
    return _sc_mean(x) + _tc_mean(x)


# speedup vs baseline: 2.5173x; 1.4515x over previous
import functools

import jax
import jax.numpy as jnp
from jax import lax
from jax.experimental import pallas as pl
from jax.experimental.pallas import tpu as pltpu
from jax.experimental.pallas import tpu_sc as plsc

B, L, D = 16, 4096, 768
LANES = 16
NCOL = D // LANES
SC_ROWS = 0
CHUNK = 64
NBUF = 2
SC_HALF = SC_ROWS // 2
NCH = SC_HALF // CHUNK
BL = 4096
TC_NL = (L - SC_ROWS) // BL
TC_OFF = SC_ROWS // BL
assert SC_ROWS % (2 * CHUNK * NBUF) == 0 or SC_ROWS == 0
assert SC_ROWS % BL == 0

_mesh = plsc.VectorSubcoreMesh(core_axis_name="c", subcore_axis_name="s")


def _zero_acc(acc):
    def body(j, _):
        acc[pl.ds(j * LANES, LANES)] = jnp.zeros((LANES,), jnp.float32)
        return 0

    lax.fori_loop(0, NCOL, body, 0)


def _accumulate(buf, acc):
    def col_body(j, _):
        col = j * LANES
        z = jnp.zeros((LANES,), jnp.float32)

        def row_body(r, carry):
            s0, s1, s2, s3 = carry
            r4 = r * 4
            s0 = s0 + buf[r4, pl.ds(col, LANES)]
            s1 = s1 + buf[r4 + 1, pl.ds(col, LANES)]
            s2 = s2 + buf[r4 + 2, pl.ds(col, LANES)]
            s3 = s3 + buf[r4 + 3, pl.ds(col, LANES)]
            return (s0, s1, s2, s3)

        s0, s1, s2, s3 = lax.fori_loop(
            0, CHUNK // 4, row_body, (z, z, z, z), unroll=4
        )
        acc[pl.ds(col, LANES)] = acc[pl.ds(col, LANES)] + (
            (s0 + s1) + (s2 + s3)
        )
        return 0

    lax.fori_loop(0, NCOL, col_body, 0)


@functools.partial(
    pl.kernel,
    mesh=_mesh,
    out_type=jax.ShapeDtypeStruct((B, 1, D), jnp.float32),
    scratch_types=[
        *([pltpu.VMEM((CHUNK, D), jnp.float32)] * NBUF),
        pltpu.VMEM((D,), jnp.float32),
        pltpu.VMEM((2, D), jnp.float32),
        pltpu.VMEM_SHARED((16, D), jnp.float32),
        *([pltpu.SemaphoreType.DMA] * NBUF),
    ],
)
def _sc_mean(x_hbm, out_hbm, *refs):
    bufs = refs[:NBUF]
    acc, pair, shared = refs[NBUF : NBUF + 3]
    sems = refs[NBUF + 3 :]
    c = lax.axis_index("c")
    s = lax.axis_index("s")
    b = c * 8 + s // 2
    row0 = (s % 2) * SC_HALF

    _zero_acc(acc)

    for i in range(NBUF):
        pltpu.async_copy(
            x_hbm.at[b, pl.ds(row0 + i * CHUNK, CHUNK)], bufs[i], sems[i]
        )

    def outer(k, _):
        for i in range(NBUF):
            idx = NBUF * k + i
            base = row0 + idx * CHUNK
            pltpu.make_async_copy(
                x_hbm.at[b, pl.ds(base, CHUNK)], bufs[i], sems[i]
            ).wait()
            _accumulate(bufs[i], acc)

            @pl.when(idx + NBUF < NCH)
            def _():
                pltpu.async_copy(
                    x_hbm.at[b, pl.ds(base + NBUF * CHUNK, CHUNK)],
                    bufs[i],
                    sems[i],
                )

        return 0

    lax.fori_loop(0, NCH // NBUF, outer, 0)

    pltpu.sync_copy(acc, shared.at[s])
    plsc.subcore_barrier()

    @pl.when(s % 2 == 0)
    def _():
        pltpu.sync_copy(shared.at[s], pair.at[0])
        pltpu.sync_copy(shared.at[s + 1], pair.at[1])

        def fin(j, _):
            col = j * LANES
            m = (pair[0, pl.ds(col, LANES)] + pair[1, pl.ds(col, LANES)]) * (
                1.0 / L
            )
            acc[pl.ds(col, LANES)] = m
            return 0

        lax.fori_loop(0, NCOL, fin, 0)
        pltpu.sync_copy(acc, out_hbm.at[b, 0])


def _tc_body(x_ref, o_ref):
    li = pl.program_id(1)

    @pl.when(li == 0)
    def _():
        o_ref[...] = jnp.zeros_like(o_ref)

    o_ref[...] += jnp.sum(x_ref[...], axis=1, keepdims=True)

    @pl.when(li == pl.num_programs(1) - 1)
    def _():
        o_ref[...] = o_ref[...] * (1.0 / L)


def _tc_mean(x):
    return pl.pallas_call(
        _tc_body,
        grid=(B, TC_NL),
        in_specs=[
            pl.BlockSpec((1, BL, D), lambda b, l: (b, l + TC_OFF, 0)),
        ],
        out_specs=pl.BlockSpec((1, 1, D), lambda b, l: (b, 0, 0)),
        out_shape=jax.ShapeDtypeStruct((B, 1, D), jnp.float32),
        compiler_params=pltpu.CompilerParams(
            dimension_semantics=("parallel", "arbitrary"),
        ),
    )(x)


def kernel(x):
    if SC_ROWS == 0:
        return _tc_mean(x)
    if SC_ROWS == L:
        return _sc_mean(x)
    return _sc_mean(x) + _tc_mean(x)
